# baseline (speedup 1.0000x reference)
import functools
import jax
import jax.numpy as jnp
from jax import lax
from jax.experimental import pallas as pl
from jax.experimental.pallas import tpu as pltpu
from jax.experimental.pallas import tpu_sc as plsc

N_ING = 100000
N_TASTE = 10000
E = 320000
D = 128
NW = 32
EPW = E // NW
CH = 80
NCHUNK = EPW // CH
NBUF = 4
NIDX = 8


def _a1_body(xt_ref, wi_ref, asrc_ref, bi_ref, wt_ref, adst_ref, bt_ref,
             vsrc_ref, csrc_ref, adstv_ref):
    a_row = asrc_ref[...]
    vsrc_ref[...] = jnp.sum(wi_ref[...] * a_row, axis=1).reshape(1, D)
    csrc_ref[...] = jnp.sum(bi_ref[...] * a_row).reshape(1, 1)
    ad_row = adst_ref[...]
    vdst = jnp.sum(wt_ref[...] * ad_row, axis=1).reshape(1, D)
    cdst = jnp.sum(bt_ref[...] * ad_row)
    adstv_ref[...] = jnp.sum(xt_ref[...] * vdst, axis=1) + cdst


def _a2_body(x_ref, v_ref, c_ref, a_ref):
    a_ref[...] = jnp.sum(x_ref[...] * v_ref[...], axis=1) + c_ref[0, 0]


def _c_body(agg_ref, den_ref, w_ref, b_ref, g_ref, be_ref, o_ref):
    A = agg_ref[0] + agg_ref[1]
    denom = (den_ref[0] + den_ref[1]).reshape(N_TASTE, 1)
    dp = denom + 1e-16
    pre = jnp.dot(A, w_ref[...], preferred_element_type=jnp.float32) / dp \
        + (denom / dp) * b_ref[...]
    out1 = jnp.maximum(pre, 0.0)
    mean = jnp.mean(out1, axis=0, keepdims=True)
    var = jnp.mean((out1 - mean) ** 2, axis=0, keepdims=True)
    o_ref[...] = jnp.maximum(
        g_ref[...] * (out1 - mean) * lax.rsqrt(var + 1e-5) + be_ref[...], 0.0)


def _edge_kernel(x_hbm, epack_hbm, asrc_hbm, adst_hbm,
                 outa_hbm, outd_hbm,
                 agg_sh, den_sh,
                 rows4_v, as4_v, ad4_v, w4_v, eidx_v,
                 gsems, ssems, isems):
    cid = lax.axis_index("c")
    sid = lax.axis_index("s")
    wid = sid * 2 + cid

    def zr(r, carry):
        for j in range(D // 16):
            rows4_v[0, r, pl.ds(j * 16, 16)] = jnp.zeros((16,), jnp.float32)
        return carry
    lax.fori_loop(0, CH, zr, 0)
    for l in range(CH // 16):
        as4_v[0, pl.ds(l * 16, 16)] = jnp.zeros((16,), jnp.float32)
    ncopies = jnp.where(sid == 15, 5, 8)

    def zc(k, carry):
        off = pl.multiple_of(sid * 640 + k * 80, 8)
        pltpu.sync_copy(rows4_v.at[0], agg_sh.at[pl.ds(off, 80)])
        pltpu.sync_copy(as4_v.at[0], den_sh.at[pl.ds(off, 80)])
        return carry
    lax.fori_loop(0, ncopies, zc, 0)
    plsc.subcore_barrier()

    tb = pl.multiple_of(wid * EPW, 8)

    def fire_idx(c):
        bi = c % NIDX
        off = pl.multiple_of(tb + c * CH, 8)
        pltpu.async_copy(epack_hbm.at[:, pl.ds(off, CH)], eidx_v.at[bi],
                         isems.at[bi])

    def launch(c):
        b = c % NBUF
        bi = c % NIDX
        pltpu.make_async_copy(epack_hbm.at[:, pl.ds(tb, CH)],
                              eidx_v.at[bi], isems.at[bi]).wait()
        pltpu.async_copy(x_hbm.at[eidx_v.at[bi, 0]], rows4_v.at[b],
                         gsems.at[b])
        pltpu.async_copy(asrc_hbm.at[eidx_v.at[bi, 0]], as4_v.at[b],
                         gsems.at[b])
        pltpu.async_copy(adst_hbm.at[eidx_v.at[bi, 1]], ad4_v.at[b],
                         gsems.at[b])

    def wait_scatter(c):
        b = c % NBUF
        bi = c % NIDX
        dst_idx = eidx_v.at[bi, 1]
        pltpu.make_async_copy(rows4_v.at[b], agg_sh.at[dst_idx],
                              ssems.at[b]).wait()
        pltpu.make_async_copy(w4_v.at[b], den_sh.at[dst_idx],
                              ssems.at[b]).wait()

    def process(c):
        b = c % NBUF
        bi = c % NIDX
        pltpu.make_async_copy(x_hbm.at[eidx_v.at[bi, 0]], rows4_v.at[b],
                              gsems.at[b]).wait()
        pltpu.make_async_copy(asrc_hbm.at[eidx_v.at[bi, 0]], as4_v.at[b],
                              gsems.at[b]).wait()
        pltpu.make_async_copy(adst_hbm.at[eidx_v.at[bi, 1]], ad4_v.at[b],
                              gsems.at[b]).wait()
        for g in range(CH // 16):
            alpha = as4_v[b, pl.ds(g * 16, 16)] + ad4_v[b, pl.ds(g * 16, 16)]
            alpha = jnp.where(alpha >= 0.0, alpha, 0.2 * alpha)
            w = jnp.exp(alpha)
            w4_v[b, pl.ds(g * 16, 16)] = w
            for e in range(16):
                ws = w[e]
                r = g * 16 + e
                for j in range(D // 16):
                    rows4_v[b, r, pl.ds(j * 16, 16)] = \
                        rows4_v[b, r, pl.ds(j * 16, 16)] * ws
        dst_idx = eidx_v.at[bi, 1]
        pltpu.async_copy(rows4_v.at[b], agg_sh.at[dst_idx], ssems.at[b],
                         add=True)
        pltpu.async_copy(w4_v.at[b], den_sh.at[dst_idx], ssems.at[b],
                         add=True)

    for c in range(4):
        fire_idx(c)
    launch(0)
    launch(1)

    def body(c, carry):
        ci = c + 4

        @pl.when(ci < NCHUNK)
        def _():
            fire_idx(ci)
        cl = c + 2

        @pl.when(cl < NCHUNK)
        def _():
            @pl.when(cl >= 4)
            def _():
                wait_scatter(cl - 4)
            launch(cl)
        process(c)
        return carry
    lax.fori_loop(0, NCHUNK, body, 0)

    def drain(c, carry):
        wait_scatter(c)
        return carry
    lax.fori_loop(NCHUNK - NBUF, NCHUNK, drain, 0)
    plsc.subcore_barrier()

    def dump(k, carry):
        off = pl.multiple_of(sid * 640 + k * 80, 8)
        pltpu.sync_copy(agg_sh.at[pl.ds(off, 80)],
                        outa_hbm.at[cid, pl.ds(off, 80)])
        pltpu.sync_copy(den_sh.at[pl.ds(off, 80)],
                        outd_hbm.at[cid, pl.ds(off, 80)])
        return carry
    lax.fori_loop(0, ncopies, dump, 0)


_edge_call = functools.partial(
    pl.kernel,
    mesh=plsc.VectorSubcoreMesh(core_axis_name="c", subcore_axis_name="s"),
    compiler_params=pltpu.CompilerParams(use_tc_tiling_on_sc=False),
    out_type=(
        jax.ShapeDtypeStruct((2, N_TASTE, D), jnp.float32),
        jax.ShapeDtypeStruct((2, N_TASTE), jnp.float32),
    ),
    scratch_types=[
        pltpu.VMEM_SHARED((N_TASTE, D), jnp.float32),
        pltpu.VMEM_SHARED((N_TASTE,), jnp.float32),
        pltpu.VMEM((NBUF, CH, D), jnp.float32),
        pltpu.VMEM((NBUF, CH), jnp.float32),
        pltpu.VMEM((NBUF, CH), jnp.float32),
        pltpu.VMEM((NBUF, CH), jnp.float32),
        pltpu.VMEM((NIDX, 2, CH), jnp.int32),
        pltpu.SemaphoreType.DMA((NBUF,)),
        pltpu.SemaphoreType.DMA((NBUF,)),
        pltpu.SemaphoreType.DMA((NIDX,)),
    ],
)(_edge_kernel)


def kernel(x_ingredient, x_taste, edge_src, edge_dst, W_ing, b_ing,
           W_taste, b_taste, att_src, att_dst, Wk, bk, q, gamma, beta):
    epack = jnp.stack([edge_src.astype(jnp.int32),
                       edge_dst.astype(jnp.int32)])

    vsrc, csrc, adstv = pl.pallas_call(
        _a1_body,
        out_shape=(
            jax.ShapeDtypeStruct((1, D), jnp.float32),
            jax.ShapeDtypeStruct((1, 1), jnp.float32),
            jax.ShapeDtypeStruct((N_TASTE,), jnp.float32),
        ),
    )(x_taste, W_ing, att_src.reshape(1, D), b_ing.reshape(1, D),
      W_taste, att_dst.reshape(1, D), b_taste.reshape(1, D))

    nblk = 20
    blk = 5120
    asrcv = pl.pallas_call(
        _a2_body,
        grid=(nblk,),
        in_specs=[
            pl.BlockSpec((blk, D), lambda i: (i, 0)),
            pl.BlockSpec((1, D), lambda i: (0, 0)),
            pl.BlockSpec((1, 1), lambda i: (0, 0)),
        ],
        out_specs=pl.BlockSpec((blk,), lambda i: (i,)),
        out_shape=jax.ShapeDtypeStruct((nblk * blk,), jnp.float32),
    )(x_ingredient, vsrc, csrc)

    agg2, den2 = _edge_call(x_ingredient, epack, asrcv, adstv)

    out = pl.pallas_call(
        _c_body,
        out_shape=jax.ShapeDtypeStruct((N_TASTE, D), jnp.float32),
    )(agg2, den2, W_ing, b_ing.reshape(1, D),
      gamma.reshape(1, D), beta.reshape(1, D))
    return out


# speedup vs baseline: 1.0754x; 1.0754x over previous
import functools
import jax
import jax.numpy as jnp
from jax import lax
from jax.experimental import pallas as pl
from jax.experimental.pallas import tpu as pltpu
from jax.experimental.pallas import tpu_sc as plsc

N_ING = 100000
N_TASTE = 10000
E = 320000
D = 128
NW = 32
EPW = E // NW
CH = 80
NCHUNK = EPW // CH
NBUF = 4
NIDX = 8


def _a1_body(xt_ref, wi_ref, asrc_ref, bi_ref, wt_ref, adst_ref, bt_ref,
             vsrc_ref, csrc_ref, adstv_ref):
    a_row = asrc_ref[...]
    vsrc_ref[...] = jnp.sum(wi_ref[...] * a_row, axis=1).reshape(1, D)
    csrc_ref[...] = jnp.sum(bi_ref[...] * a_row).reshape(1, 1)
    ad_row = adst_ref[...]
    vdst = jnp.sum(wt_ref[...] * ad_row, axis=1).reshape(1, D)
    cdst = jnp.sum(bt_ref[...] * ad_row)
    adstv_ref[...] = jnp.sum(xt_ref[...] * vdst, axis=1) + cdst


def _a2_body(x_ref, v_ref, c_ref, a_ref):
    a_ref[...] = jnp.sum(x_ref[...] * v_ref[...], axis=1) + c_ref[0, 0]


def _c_body(agg_ref, den_ref, w_ref, b_ref, g_ref, be_ref, o_ref):
    A = agg_ref[0] + agg_ref[1]
    denom = (den_ref[0] + den_ref[1]).reshape(N_TASTE, 1)
    dp = denom + 1e-16
    pre = jnp.dot(A, w_ref[...], preferred_element_type=jnp.float32) / dp \
        + (denom / dp) * b_ref[...]
    out1 = jnp.maximum(pre, 0.0)
    mean = jnp.mean(out1, axis=0, keepdims=True)
    var = jnp.mean((out1 - mean) ** 2, axis=0, keepdims=True)
    o_ref[...] = jnp.maximum(
        g_ref[...] * (out1 - mean) * lax.rsqrt(var + 1e-5) + be_ref[...], 0.0)


def _edge_kernel(x_hbm, esrc_hbm, edst_hbm, asrc_hbm, adst_hbm,
                 outa_hbm, outd_hbm,
                 agg_sh, den_sh,
                 rows4_v, as4_v, ad4_v, w4_v, sidx_v, didx_v,
                 gsems, ssems, isems):
    cid = lax.axis_index("c")
    sid = lax.axis_index("s")
    wid = sid * 2 + cid

    def zr(r, carry):
        for j in range(D // 16):
            rows4_v[0, r, pl.ds(j * 16, 16)] = jnp.zeros((16,), jnp.float32)
        return carry
    lax.fori_loop(0, CH, zr, 0)
    for l in range(CH // 16):
        as4_v[0, pl.ds(l * 16, 16)] = jnp.zeros((16,), jnp.float32)
    ncopies = jnp.where(sid == 15, 5, 8)

    def zc(k, carry):
        off = pl.multiple_of(sid * 640 + k * 80, 8)
        pltpu.sync_copy(rows4_v.at[0], agg_sh.at[pl.ds(off, 80)])
        pltpu.sync_copy(as4_v.at[0], den_sh.at[pl.ds(off, 80)])
        return carry
    lax.fori_loop(0, ncopies, zc, 0)
    plsc.subcore_barrier()

    tb = pl.multiple_of(wid * EPW, 8)

    def fire_idx(c):
        bi = c % NIDX
        off = pl.multiple_of(tb + c * CH, 8)
        pltpu.async_copy(esrc_hbm.at[pl.ds(off, CH)], sidx_v.at[bi],
                         isems.at[bi])
        pltpu.async_copy(edst_hbm.at[pl.ds(off, CH)], didx_v.at[bi],
                         isems.at[bi])

    def launch(c):
        b = c % NBUF
        bi = c % NIDX
        pltpu.make_async_copy(esrc_hbm.at[pl.ds(tb, CH)], sidx_v.at[bi],
                              isems.at[bi]).wait()
        pltpu.make_async_copy(edst_hbm.at[pl.ds(tb, CH)], didx_v.at[bi],
                              isems.at[bi]).wait()
        pltpu.async_copy(x_hbm.at[sidx_v.at[bi]], rows4_v.at[b],
                         gsems.at[b])
        pltpu.async_copy(asrc_hbm.at[sidx_v.at[bi]], as4_v.at[b],
                         gsems.at[b])
        pltpu.async_copy(adst_hbm.at[didx_v.at[bi]], ad4_v.at[b],
                         gsems.at[b])

    def wait_scatter(c):
        b = c % NBUF
        bi = c % NIDX
        dst_idx = didx_v.at[bi]
        pltpu.make_async_copy(rows4_v.at[b], agg_sh.at[dst_idx],
                              ssems.at[b]).wait()
        pltpu.make_async_copy(w4_v.at[b], den_sh.at[dst_idx],
                              ssems.at[b]).wait()

    def process(c):
        b = c % NBUF
        bi = c % NIDX
        pltpu.make_async_copy(x_hbm.at[sidx_v.at[bi]], rows4_v.at[b],
                              gsems.at[b]).wait()
        pltpu.make_async_copy(asrc_hbm.at[sidx_v.at[bi]], as4_v.at[b],
                              gsems.at[b]).wait()
        pltpu.make_async_copy(adst_hbm.at[didx_v.at[bi]], ad4_v.at[b],
                              gsems.at[b]).wait()
        for g in range(CH // 16):
            alpha = as4_v[b, pl.ds(g * 16, 16)] + ad4_v[b, pl.ds(g * 16, 16)]
            alpha = jnp.where(alpha >= 0.0, alpha, 0.2 * alpha)
            w = jnp.exp(alpha)
            w4_v[b, pl.ds(g * 16, 16)] = w
            for e in range(16):
                ws = w[e]
                r = g * 16 + e
                for j in range(D // 16):
                    rows4_v[b, r, pl.ds(j * 16, 16)] = \
                        rows4_v[b, r, pl.ds(j * 16, 16)] * ws
        dst_idx = didx_v.at[bi]
        pltpu.async_copy(rows4_v.at[b], agg_sh.at[dst_idx], ssems.at[b],
                         add=True)
        pltpu.async_copy(w4_v.at[b], den_sh.at[dst_idx], ssems.at[b],
                         add=True)

    for c in range(4):
        fire_idx(c)
    launch(0)
    launch(1)

    def body(c, carry):
        ci = c + 4

        @pl.when(ci < NCHUNK)
        def _():
            fire_idx(ci)
        cl = c + 2

        @pl.when(cl < NCHUNK)
        def _():
            @pl.when(cl >= 4)
            def _():
                wait_scatter(cl - 4)
            launch(cl)
        process(c)
        return carry
    lax.fori_loop(0, NCHUNK, body, 0)

    def drain(c, carry):
        wait_scatter(c)
        return carry
    lax.fori_loop(NCHUNK - NBUF, NCHUNK, drain, 0)
    plsc.subcore_barrier()

    def dump(k, carry):
        off = pl.multiple_of(sid * 640 + k * 80, 8)
        pltpu.sync_copy(agg_sh.at[pl.ds(off, 80)],
                        outa_hbm.at[cid, pl.ds(off, 80)])
        pltpu.sync_copy(den_sh.at[pl.ds(off, 80)],
                        outd_hbm.at[cid, pl.ds(off, 80)])
        return carry
    lax.fori_loop(0, ncopies, dump, 0)


_edge_call = functools.partial(
    pl.kernel,
    mesh=plsc.VectorSubcoreMesh(core_axis_name="c", subcore_axis_name="s"),
    compiler_params=pltpu.CompilerParams(use_tc_tiling_on_sc=False),
    out_type=(
        jax.ShapeDtypeStruct((2, N_TASTE, D), jnp.float32),
        jax.ShapeDtypeStruct((2, N_TASTE), jnp.float32),
    ),
    scratch_types=[
        pltpu.VMEM_SHARED((N_TASTE, D), jnp.float32),
        pltpu.VMEM_SHARED((N_TASTE,), jnp.float32),
        pltpu.VMEM((NBUF, CH, D), jnp.float32),
        pltpu.VMEM((NBUF, CH), jnp.float32),
        pltpu.VMEM((NBUF, CH), jnp.float32),
        pltpu.VMEM((NBUF, CH), jnp.float32),
        pltpu.VMEM((NIDX, CH), jnp.int32),
        pltpu.VMEM((NIDX, CH), jnp.int32),
        pltpu.SemaphoreType.DMA((NBUF,)),
        pltpu.SemaphoreType.DMA((NBUF,)),
        pltpu.SemaphoreType.DMA((NIDX,)),
    ],
)(_edge_kernel)


def kernel(x_ingredient, x_taste, edge_src, edge_dst, W_ing, b_ing,
           W_taste, b_taste, att_src, att_dst, Wk, bk, q, gamma, beta):
    esrc = edge_src.astype(jnp.int32)
    edst = edge_dst.astype(jnp.int32)

    vsrc, csrc, adstv = pl.pallas_call(
        _a1_body,
        out_shape=(
            jax.ShapeDtypeStruct((1, D), jnp.float32),
            jax.ShapeDtypeStruct((1, 1), jnp.float32),
            jax.ShapeDtypeStruct((N_TASTE,), jnp.float32),
        ),
    )(x_taste, W_ing, att_src.reshape(1, D), b_ing.reshape(1, D),
      W_taste, att_dst.reshape(1, D), b_taste.reshape(1, D))

    nblk = 20
    blk = 5120
    asrcv = pl.pallas_call(
        _a2_body,
        grid=(nblk,),
        in_specs=[
            pl.BlockSpec((blk, D), lambda i: (i, 0)),
            pl.BlockSpec((1, D), lambda i: (0, 0)),
            pl.BlockSpec((1, 1), lambda i: (0, 0)),
        ],
        out_specs=pl.BlockSpec((blk,), lambda i: (i,)),
        out_shape=jax.ShapeDtypeStruct((nblk * blk,), jnp.float32),
    )(x_ingredient, vsrc, csrc)

    agg2, den2 = _edge_call(x_ingredient, esrc, edst, asrcv, adstv)

    out = pl.pallas_call(
        _c_body,
        out_shape=jax.ShapeDtypeStruct((N_TASTE, D), jnp.float32),
    )(agg2, den2, W_ing, b_ing.reshape(1, D),
      gamma.reshape(1, D), beta.reshape(1, D))
    return out
